# baseline (speedup 1.0000x reference)
import functools

import jax
import jax.numpy as jnp
from jax import lax
from jax.experimental import pallas as pl
from jax.experimental.pallas import tpu as pltpu
from jax.experimental.pallas import tpu_sc as plsc


def _argmin_body(x_ref, c_ref, w_ref, b_ref, out_ref, cproj_ref, csq_ref,
                 *, tm, tk, nk, tp):
    i = pl.program_id(0)
    nj = nk // tk
    nc = tk // 128
    npj = nk // tp

    @pl.when(i == 0)
    def _():
        for j in range(nj):
            cj = c_ref[pl.ds(j * tk, tk), :]
            csq_ref[:, pl.ds(j * tk, tk)] = jnp.sum(cj * cj, axis=1)[None, :]

    @pl.when(i < npj)
    def _():
        base = jnp.minimum(i, npj - 1) * tp
        cp = c_ref[pl.ds(base, tp), :]
        cproj_ref[...] = lax.dot_general(
            cp, w_ref[...], (((1,), (1,)), ((), ())),
            preferred_element_type=jnp.float32) + b_ref[...]

    x = x_ref[...]
    x_sq = jnp.sum(x * x, axis=1, keepdims=True)

    v = jnp.full((tm, 128), jnp.inf, jnp.float32)
    a = jnp.zeros((tm, 128), jnp.int32)
    for j in range(nj):
        c = c_ref[pl.ds(j * tk, tk), :]
        c_sq = csq_ref[:, pl.ds(j * tk, tk)]
        cross = lax.dot_general(x, c, (((1,), (1,)), ((), ())),
                                preferred_element_type=jnp.float32)
        d2 = (x_sq + c_sq) - 2.0 * cross
        for ch in range(nc):
            blk = d2[:, ch * 128:(ch + 1) * 128]
            upd = blk < v
            v = jnp.where(upd, blk, v)
            a = jnp.where(upd, jnp.int32(j * nc + ch), a)

    m = jnp.min(v, axis=1, keepdims=True)
    s = jnp.sqrt(m)
    mb = lax.bitcast_convert_type(m, jnp.int32)
    hi = m
    for u in range(1, 6):
        t = lax.bitcast_convert_type(mb + u, jnp.float32)
        hi = jnp.where(jnp.sqrt(t) == s, t, hi)

    lane = lax.broadcasted_iota(jnp.int32, (tm, 128), 1)
    g = a * 128 + lane
    arg = jnp.min(jnp.where(v <= hi, g, jnp.int32(1 << 30)),
                  axis=1, keepdims=True)
    out_ref[...] = arg


def _nearest_cluster_and_proj(xf, centers, w, bias, *, tm, tk):
    m, d = xf.shape
    k = centers.shape[0]
    ni = m // tm
    npj = 1
    while npj * 2 <= ni and npj * 2 <= k:
        npj *= 2
    tp = k // npj
    sel, cproj = pl.pallas_call(
        functools.partial(_argmin_body, tm=tm, tk=tk, nk=k, tp=tp),
        grid=(ni,),
        in_specs=[
            pl.BlockSpec((tm, d), lambda i: (i, 0)),
            pl.BlockSpec((k, d), lambda i: (0, 0)),
            pl.BlockSpec((d, d), lambda i: (0, 0)),
            pl.BlockSpec((1, d), lambda i: (0, 0)),
        ],
        out_specs=[
            pl.BlockSpec((tm, 1), lambda i: (i, 0)),
            pl.BlockSpec((tp, d), lambda i: (jnp.minimum(i, k // tp - 1), 0)),
        ],
        out_shape=[
            jax.ShapeDtypeStruct((m, 1), jnp.int32),
            jax.ShapeDtypeStruct((k, d), jnp.float32),
        ],
        scratch_shapes=[pltpu.VMEM((1, k), jnp.float32)],
    )(xf, centers, w, bias.reshape(1, d))
    return sel.reshape(m), cproj


def _make_sc_gather(v, d, b):
    info = plsc.get_sparse_core_info()
    nw = info.num_cores * info.num_subcores
    assert b % (8 * nw) == 0
    b_per_w = b // nw
    mesh = plsc.VectorSubcoreMesh(core_axis_name="c", subcore_axis_name="s")

    @functools.partial(
        pl.kernel, mesh=mesh,
        out_type=jax.ShapeDtypeStruct((b, d), jnp.float32),
        scratch_types=[
            pltpu.VMEM((b_per_w,), jnp.int32),
            pltpu.VMEM((b_per_w, d), jnp.float32),
            pltpu.SemaphoreType.DMA,
        ],
    )
    def gather(table_hbm, idx_hbm, out_hbm, idx_v, rows_v, sem):
        wid = lax.axis_index("s") * info.num_cores + lax.axis_index("c")
        base = wid * b_per_w
        pltpu.sync_copy(idx_hbm.at[pl.ds(base, b_per_w)], idx_v)
        pltpu.async_copy(table_hbm.at[idx_v], rows_v, sem).wait()
        pltpu.sync_copy(rows_v, out_hbm.at[pl.ds(base, b_per_w)])

    return gather


def kernel(x, cluster_centers, W, b):
    bb, s, d = x.shape
    m = bb * s
    xf = x.reshape(m, d)
    selected, cproj = _nearest_cluster_and_proj(
        xf, cluster_centers, W, b, tm=128, tk=2048)
    out = _make_sc_gather(cluster_centers.shape[0], d, m)(cproj, selected)
    return out.reshape(bb, s, W.shape[0])


# speedup vs baseline: 1.3441x; 1.3441x over previous
import functools

import jax
import jax.numpy as jnp
from jax import lax
from jax.experimental import pallas as pl
from jax.experimental.pallas import tpu as pltpu
from jax.experimental.pallas import tpu_sc as plsc


def _argmin_body(x_ref, c_ref, w_ref, b_ref, out_ref, cproj_ref, csq_ref,
                 *, tm, tk, nk, tp):
    i = pl.program_id(0)
    nj = nk // tk
    nc = tk // 128
    npj = nk // tp

    @pl.when(i == 0)
    def _():
        for j in range(nj):
            cj = c_ref[pl.ds(j * tk, tk), :]
            csq_ref[:, pl.ds(j * tk, tk)] = jnp.sum(cj * cj, axis=1)[None, :]

    @pl.when(i < npj)
    def _():
        base = jnp.minimum(i, npj - 1) * tp
        cp = c_ref[pl.ds(base, tp), :]
        cproj_ref[...] = lax.dot_general(
            cp, w_ref[...], (((1,), (1,)), ((), ())),
            preferred_element_type=jnp.float32) + b_ref[...]

    x = x_ref[...]
    x_sq = jnp.sum(x * x, axis=1, keepdims=True)

    v = jnp.full((tm, 128), jnp.inf, jnp.float32)
    a = jnp.zeros((tm, 128), jnp.int32)
    for j in range(nj):
        c = c_ref[pl.ds(j * tk, tk), :]
        c_sq = csq_ref[:, pl.ds(j * tk, tk)]
        cross = lax.dot_general(x, c, (((1,), (1,)), ((), ())),
                                preferred_element_type=jnp.float32)
        d2 = (x_sq + c_sq) - 2.0 * cross
        for ch in range(nc):
            blk = d2[:, ch * 128:(ch + 1) * 128]
            upd = blk < v
            v = jnp.where(upd, blk, v)
            a = jnp.where(upd, jnp.int32(j * nc + ch), a)

    m = jnp.min(v, axis=1, keepdims=True)
    s = jnp.sqrt(m)
    mb = lax.bitcast_convert_type(m, jnp.int32)
    hi = m
    for u in range(1, 6):
        t = lax.bitcast_convert_type(mb + u, jnp.float32)
        hi = jnp.where(jnp.sqrt(t) == s, t, hi)

    lane = lax.broadcasted_iota(jnp.int32, (tm, 128), 1)
    g = a * 128 + lane
    arg = jnp.min(jnp.where(v <= hi, g, jnp.int32(1 << 30)),
                  axis=1, keepdims=True)
    out_ref[...] = arg


def _nearest_cluster_and_proj(xf, centers, w, bias, *, tm, tk):
    m, d = xf.shape
    k = centers.shape[0]
    ni = m // tm
    npj = 1
    while npj * 2 <= ni and npj * 2 <= k:
        npj *= 2
    tp = k // npj
    sel, cproj = pl.pallas_call(
        functools.partial(_argmin_body, tm=tm, tk=tk, nk=k, tp=tp),
        grid=(ni,),
        in_specs=[
            pl.BlockSpec((tm, d), lambda i: (i, 0)),
            pl.BlockSpec((k, d), lambda i: (0, 0)),
            pl.BlockSpec((d, d), lambda i: (0, 0)),
            pl.BlockSpec((1, d), lambda i: (0, 0)),
        ],
        out_specs=[
            pl.BlockSpec((tm, 1), lambda i: (i, 0)),
            pl.BlockSpec((tp, d), lambda i: (jnp.minimum(i, k // tp - 1), 0)),
        ],
        out_shape=[
            jax.ShapeDtypeStruct((m, 1), jnp.int32),
            jax.ShapeDtypeStruct((k, d), jnp.float32),
        ],
        scratch_shapes=[pltpu.VMEM((1, k), jnp.float32)],
    )(xf, centers, w, bias.reshape(1, d))
    return sel.reshape(m), cproj


def _make_sc_gather(v, d, b):
    info = plsc.get_sparse_core_info()
    nw = info.num_cores * info.num_subcores
    assert b % (8 * nw) == 0
    b_per_w = b // nw
    mesh = plsc.VectorSubcoreMesh(core_axis_name="c", subcore_axis_name="s")

    h = b_per_w // 2
    assert h % 8 == 0

    @functools.partial(
        pl.kernel, mesh=mesh,
        out_type=jax.ShapeDtypeStruct((b, d), jnp.float32),
        scratch_types=[
            pltpu.VMEM((b_per_w,), jnp.int32),
            pltpu.VMEM((b_per_w, d), jnp.float32),
            pltpu.SemaphoreType.DMA,
            pltpu.SemaphoreType.DMA,
            pltpu.SemaphoreType.DMA,
            pltpu.SemaphoreType.DMA,
        ],
    )
    def gather(table_hbm, idx_hbm, out_hbm, idx_v, rows_v, s0, s1, s2, s3):
        wid = lax.axis_index("s") * info.num_cores + lax.axis_index("c")
        base = wid * b_per_w
        pltpu.sync_copy(idx_hbm.at[pl.ds(base, b_per_w)], idx_v)
        g0 = pltpu.async_copy(table_hbm.at[idx_v.at[pl.ds(0, h)]],
                              rows_v.at[pl.ds(0, h)], s0)
        g1 = pltpu.async_copy(table_hbm.at[idx_v.at[pl.ds(h, h)]],
                              rows_v.at[pl.ds(h, h)], s1)
        g0.wait()
        o0 = pltpu.async_copy(rows_v.at[pl.ds(0, h)],
                              out_hbm.at[pl.ds(base, h)], s2)
        g1.wait()
        o1 = pltpu.async_copy(rows_v.at[pl.ds(h, h)],
                              out_hbm.at[pl.ds(base + h, h)], s3)
        o0.wait()
        o1.wait()

    return gather


def kernel(x, cluster_centers, W, b):
    bb, s, d = x.shape
    m = bb * s
    xf = x.reshape(m, d)
    selected, cproj = _nearest_cluster_and_proj(
        xf, cluster_centers, W, b, tm=256, tk=2048)
    out = _make_sc_gather(cluster_centers.shape[0], d, m)(cproj, selected)
    return out.reshape(bb, s, W.shape[0])


# speedup vs baseline: 1.3481x; 1.0029x over previous
import functools

import jax
import jax.numpy as jnp
from jax import lax
from jax.experimental import pallas as pl
from jax.experimental.pallas import tpu as pltpu
from jax.experimental.pallas import tpu_sc as plsc


def _argmin_body(x_ref, c_ref, w_ref, b_ref, out_ref, cproj_ref, csq_ref,
                 *, tm, tk, nk, tp):
    i = pl.program_id(0)
    nj = nk // tk
    nc = tk // 128
    npj = nk // tp

    @pl.when(i == 0)
    def _():
        for j in range(nj):
            cj = c_ref[pl.ds(j * tk, tk), :]
            csq_ref[:, pl.ds(j * tk, tk)] = jnp.sum(cj * cj, axis=1)[None, :]

    @pl.when(i < npj)
    def _():
        base = jnp.minimum(i, npj - 1) * tp
        cp = c_ref[pl.ds(base, tp), :]
        cproj_ref[...] = lax.dot_general(
            cp, w_ref[...], (((1,), (1,)), ((), ())),
            preferred_element_type=jnp.float32) + b_ref[...]

    x = x_ref[...]
    x_sq = jnp.sum(x * x, axis=1, keepdims=True)

    v = jnp.full((tm, 128), jnp.inf, jnp.float32)
    a = jnp.zeros((tm, 128), jnp.int32)
    for j in range(nj):
        c = c_ref[pl.ds(j * tk, tk), :]
        c_sq = csq_ref[:, pl.ds(j * tk, tk)]
        cross = lax.dot_general(x, c, (((1,), (1,)), ((), ())),
                                preferred_element_type=jnp.float32)
        d2 = (x_sq + c_sq) - 2.0 * cross
        for ch in range(nc):
            blk = d2[:, ch * 128:(ch + 1) * 128]
            upd = blk < v
            v = jnp.where(upd, blk, v)
            a = jnp.where(upd, jnp.int32(j * nc + ch), a)

    m = jnp.min(v, axis=1, keepdims=True)
    s = jnp.sqrt(m)
    mb = lax.bitcast_convert_type(m, jnp.int32)
    hi = m
    for u in range(1, 6):
        t = lax.bitcast_convert_type(mb + u, jnp.float32)
        hi = jnp.where(jnp.sqrt(t) == s, t, hi)

    lane = lax.broadcasted_iota(jnp.int32, (tm, 128), 1)
    g = a * 128 + lane
    arg = jnp.min(jnp.where(v <= hi, g, jnp.int32(1 << 30)),
                  axis=1, keepdims=True)
    out_ref[...] = arg


def _nearest_cluster_and_proj(xf, centers, w, bias, *, tm, tk):
    m, d = xf.shape
    k = centers.shape[0]
    ni = m // tm
    npj = 1
    while npj * 2 <= ni and npj * 2 <= k:
        npj *= 2
    tp = k // npj
    sel, cproj = pl.pallas_call(
        functools.partial(_argmin_body, tm=tm, tk=tk, nk=k, tp=tp),
        grid=(ni,),
        in_specs=[
            pl.BlockSpec((tm, d), lambda i: (i, 0)),
            pl.BlockSpec((k, d), lambda i: (0, 0)),
            pl.BlockSpec((d, d), lambda i: (0, 0)),
            pl.BlockSpec((1, d), lambda i: (0, 0)),
        ],
        out_specs=[
            pl.BlockSpec((tm, 1), lambda i: (i, 0)),
            pl.BlockSpec((tp, d), lambda i: (jnp.minimum(i, k // tp - 1), 0)),
        ],
        out_shape=[
            jax.ShapeDtypeStruct((m, 1), jnp.int32),
            jax.ShapeDtypeStruct((k, d), jnp.float32),
        ],
        scratch_shapes=[pltpu.VMEM((1, k), jnp.float32)],
    )(xf, centers, w, bias.reshape(1, d))
    return sel.reshape(m), cproj


def _make_sc_gather(v, d, b):
    info = plsc.get_sparse_core_info()
    nw = info.num_cores * info.num_subcores
    assert b % (8 * nw) == 0
    b_per_w = b // nw
    mesh = plsc.VectorSubcoreMesh(core_axis_name="c", subcore_axis_name="s")

    @functools.partial(
        pl.kernel, mesh=mesh,
        out_type=jax.ShapeDtypeStruct((b, d), jnp.float32),
        scratch_types=[
            pltpu.VMEM((b_per_w,), jnp.int32),
            pltpu.VMEM((b_per_w, d), jnp.float32),
            pltpu.SemaphoreType.DMA,
        ],
    )
    def gather(table_hbm, idx_hbm, out_hbm, idx_v, rows_v, sem):
        wid = lax.axis_index("s") * info.num_cores + lax.axis_index("c")
        base = wid * b_per_w
        pltpu.sync_copy(idx_hbm.at[pl.ds(base, b_per_w)], idx_v)
        pltpu.async_copy(table_hbm.at[idx_v], rows_v, sem).wait()
        pltpu.sync_copy(rows_v, out_hbm.at[pl.ds(base, b_per_w)])

    return gather


def kernel(x, cluster_centers, W, b):
    bb, s, d = x.shape
    m = bb * s
    xf = x.reshape(m, d)
    selected, cproj = _nearest_cluster_and_proj(
        xf, cluster_centers, W, b, tm=256, tk=4096)
    out = _make_sc_gather(cluster_centers.shape[0], d, m)(cproj, selected)
    return out.reshape(bb, s, W.shape[0])


# speedup vs baseline: 1.3540x; 1.0044x over previous
import functools

import jax
import jax.numpy as jnp
from jax import lax
from jax.experimental import pallas as pl
from jax.experimental.pallas import tpu as pltpu
from jax.experimental.pallas import tpu_sc as plsc


def _argmin_body(x_ref, c_ref, w_ref, b_ref, out_ref, cproj_ref, csq_ref,
                 *, tm, tk, nk, tp):
    i = pl.program_id(0)
    nj = nk // tk
    nc = tk // 128
    npj = nk // tp

    @pl.when(i == 0)
    def _():
        for j in range(nj):
            cj = c_ref[pl.ds(j * tk, tk), :]
            csq_ref[:, pl.ds(j * tk, tk)] = jnp.sum(cj * cj, axis=1)[None, :]

    @pl.when(i < npj)
    def _():
        base = jnp.minimum(i, npj - 1) * tp
        cp = c_ref[pl.ds(base, tp), :]
        cproj_ref[...] = lax.dot_general(
            cp, w_ref[...], (((1,), (1,)), ((), ())),
            preferred_element_type=jnp.float32) + b_ref[...]

    x = x_ref[...]
    x_sq = jnp.sum(x * x, axis=1, keepdims=True)

    v = jnp.full((tm, 128), jnp.inf, jnp.float32)
    a = jnp.zeros((tm, 128), jnp.int32)
    for j in range(nj):
        c = c_ref[pl.ds(j * tk, tk), :]
        c_sq = csq_ref[:, pl.ds(j * tk, tk)]
        cross = lax.dot_general(x, c, (((1,), (1,)), ((), ())),
                                preferred_element_type=jnp.float32)
        d2 = (x_sq + c_sq) - 2.0 * cross
        for ch in range(nc):
            blk = d2[:, ch * 128:(ch + 1) * 128]
            upd = blk < v
            v = jnp.where(upd, blk, v)
            a = jnp.where(upd, jnp.int32(j * nc + ch), a)

    m = jnp.min(v, axis=1, keepdims=True)
    s = jnp.sqrt(m)
    mb = lax.bitcast_convert_type(m, jnp.int32)
    hi = m
    for u in range(1, 6):
        t = lax.bitcast_convert_type(mb + u, jnp.float32)
        hi = jnp.where(jnp.sqrt(t) == s, t, hi)

    lane = lax.broadcasted_iota(jnp.int32, (tm, 128), 1)
    g = a * 128 + lane
    arg = jnp.min(jnp.where(v <= hi, g, jnp.int32(1 << 30)),
                  axis=1, keepdims=True)
    out_ref[...] = arg


def _nearest_cluster_and_proj(xf, centers, w, bias, *, tm, tk):
    m, d = xf.shape
    k = centers.shape[0]
    ni = m // tm
    npj = 1
    while npj * 2 <= ni and npj * 2 <= k:
        npj *= 2
    tp = k // npj
    sel, cproj = pl.pallas_call(
        functools.partial(_argmin_body, tm=tm, tk=tk, nk=k, tp=tp),
        grid=(ni,),
        in_specs=[
            pl.BlockSpec((tm, d), lambda i: (i, 0)),
            pl.BlockSpec((k, d), lambda i: (0, 0)),
            pl.BlockSpec((d, d), lambda i: (0, 0)),
            pl.BlockSpec((1, d), lambda i: (0, 0)),
        ],
        out_specs=[
            pl.BlockSpec((tm, 1), lambda i: (i, 0)),
            pl.BlockSpec((tp, d), lambda i: (jnp.minimum(i, k // tp - 1), 0)),
        ],
        out_shape=[
            jax.ShapeDtypeStruct((m, 1), jnp.int32),
            jax.ShapeDtypeStruct((k, d), jnp.float32),
        ],
        scratch_shapes=[pltpu.VMEM((1, k), jnp.float32)],
    )(xf, centers, w, bias.reshape(1, d))
    return sel.reshape(m), cproj


def _make_sc_gather(v, d, b):
    info = plsc.get_sparse_core_info()
    nw = info.num_cores * info.num_subcores
    assert b % (8 * nw) == 0
    b_per_w = b // nw
    mesh = plsc.VectorSubcoreMesh(core_axis_name="c", subcore_axis_name="s")

    @functools.partial(
        pl.kernel, mesh=mesh,
        out_type=jax.ShapeDtypeStruct((b, d), jnp.float32),
        scratch_types=[
            pltpu.VMEM((b_per_w,), jnp.int32),
            pltpu.VMEM((b_per_w, d), jnp.float32),
            pltpu.SemaphoreType.DMA,
        ],
    )
    def gather(table_hbm, idx_hbm, out_hbm, idx_v, rows_v, sem):
        wid = lax.axis_index("s") * info.num_cores + lax.axis_index("c")
        base = wid * b_per_w
        pltpu.sync_copy(idx_hbm.at[pl.ds(base, b_per_w)], idx_v)
        pltpu.async_copy(table_hbm.at[idx_v], rows_v, sem).wait()
        pltpu.sync_copy(rows_v, out_hbm.at[pl.ds(base, b_per_w)])

    return gather


def kernel(x, cluster_centers, W, b):
    bb, s, d = x.shape
    m = bb * s
    xf = x.reshape(m, d)
    selected, cproj = _nearest_cluster_and_proj(
        xf, cluster_centers, W, b, tm=256, tk=2048)
    out = _make_sc_gather(cluster_centers.shape[0], d, m)(cproj, selected)
    return out.reshape(bb, s, W.shape[0])
